# Initial kernel scaffold; baseline (speedup 1.0000x reference)
import jax
import jax.numpy as jnp
from jax.experimental import pallas as pl


def kernel(feats, points0, neighbors0, points1, neighbors1, points2, neighbors2, points3, neighbors3, subsampling0, subsampling1, subsampling2, ups1, ups2, W_e11, W1a, W1b, Wsc1, W2a, W2b, W3a, W3b, W4a, W4b, W5a, W5b, W6a, W6b, Wsc6, W7a, W7b, Wsc7, Wd3, Wd2, Wq1, Wk1, Wv1, lin1W, lin1b, Wq2, Wk2, Wv2, lin2W, lin2b, Wq3, Wk3, Wv3, lin3W, lin3b):
    raise NotImplementedError("write your pallas kernel here")


# speedup vs baseline: 1.1859x; 1.1859x over previous
import functools

import jax
import jax.numpy as jnp
from jax import lax
from jax.experimental import pallas as pl
from jax.experimental.pallas import tpu as pltpu
from jax.experimental.pallas import tpu_sc as plsc

_NC, _NS = 2, 16
_NW = _NC * _NS
_L = 16


def _gn_mat(c, c_pad=None):
    g = min(32, c)
    s = c // g
    i = jnp.arange(c)
    M = jnp.asarray(i[:, None] // s == i[None, :] // s, jnp.float32) / s
    if c_pad is not None and c_pad != c:
        M = jnp.pad(M, ((0, c_pad - c), (0, c_pad - c)))
    return M


def _cpad(W, to=128):
    return jnp.pad(W, ((0, 0), (0, to - W.shape[1])))


def _rpad(W, to=128):
    return jnp.pad(W, ((0, to - W.shape[0]), (0, 0)))


def _tc_stage(x, W=None, skip=None, skipW=None, M=None, gn_post=False,
              act=False):
    N = x.shape[0]
    Cout = W.shape[1] if W is not None else x.shape[1]
    BR = N if N <= 512 else 512
    grid = ((N + BR - 1) // BR,)

    ops = [x]
    specs = [pl.BlockSpec((BR, x.shape[1]), lambda i: (i, 0))]
    if W is not None:
        ops.append(W)
        specs.append(pl.BlockSpec(W.shape, lambda i: (0, 0)))
    if skip is not None:
        ops.append(skip)
        specs.append(pl.BlockSpec((BR, skip.shape[1]), lambda i: (i, 0)))
    if skipW is not None:
        ops.append(skipW)
        specs.append(pl.BlockSpec(skipW.shape, lambda i: (0, 0)))
    if M is not None:
        ops.append(M)
        specs.append(pl.BlockSpec(M.shape, lambda i: (0, 0)))

    has_W, has_skip, has_skipW = W is not None, skip is not None, skipW is not None
    has_M = M is not None

    def body(*refs):
        out_ref = refs[-1]
        it = iter(refs[:-1])
        z = next(it)[...]
        if has_W:
            z = jnp.dot(z, next(it)[...], preferred_element_type=jnp.float32)
        s = None
        if has_skip:
            s = next(it)[...]
            if has_skipW:
                s = jnp.dot(s, next(it)[...], preferred_element_type=jnp.float32)
        if has_M:
            Mv = next(it)[...]
            if gn_post and s is not None:
                z = z + s
                s = None
            mean = jnp.dot(z, Mv, preferred_element_type=jnp.float32,
                           precision=lax.Precision.HIGHEST)
            zc = z - mean
            var = jnp.dot(zc * zc, Mv, preferred_element_type=jnp.float32,
                          precision=lax.Precision.HIGHEST)
            z = zc * lax.rsqrt(var + 1e-5)
        if s is not None:
            z = z + s
        if act:
            z = jnp.where(z > 0, z, 0.1 * z)
        out_ref[...] = z

    return pl.pallas_call(
        body,
        grid=grid,
        in_specs=specs,
        out_specs=pl.BlockSpec((BR, Cout), lambda i: (i, 0)),
        out_shape=jax.ShapeDtypeStruct((N, Cout), jnp.float32),
    )(*ops)


def _sc_gather_max(table, idx, R, D_real=None):
    Nout, K = idx.shape
    D = table.shape[1]
    if D_real is None:
        D_real = D
    gran = _NW * R
    Npad = -(-Nout // gran) * gran
    rows_w = Npad // _NW
    chunks = rows_w // R
    width = min(128, R * K)
    J = (R * K) // width
    flat = jnp.pad(idx.reshape(-1), (0, (Npad - Nout) * K))

    mesh = plsc.VectorSubcoreMesh(core_axis_name="c", subcore_axis_name="s")

    @functools.partial(
        pl.kernel,
        out_type=jax.ShapeDtypeStruct((Npad, D), jnp.float32),
        mesh=mesh,
        scratch_types=[
            pltpu.VMEM((R * K,), jnp.int32),
            pltpu.VMEM((R * K, D), jnp.float32),
            pltpu.VMEM((R, D), jnp.float32),
            pltpu.SemaphoreType.DMA,
        ],
    )
    def k(table_h, idx_h, out_h, idx_v, gbuf, outv, sem):
        wid = lax.axis_index("s") * _NC + lax.axis_index("c")
        row0w = wid * rows_w

        if K != 1 and D_real < D:
            zeros = jnp.zeros((_L,), jnp.float32)
            for r in range(R):
                for c in range(D_real // _L, D // _L):
                    outv[r, pl.ds(c * _L, _L)] = zeros

        def chunk_body(ci, carry):
            row0 = row0w + ci * R
            pltpu.sync_copy(idx_h.at[pl.ds(row0 * K, R * K)], idx_v)
            cps = [
                pltpu.async_copy(table_h.at[idx_v.at[pl.ds(j * width, width)]],
                                 gbuf.at[pl.ds(j * width, width)], sem)
                for j in range(J)
            ]
            for cp in cps:
                cp.wait()
            if K == 1:
                pltpu.sync_copy(gbuf, out_h.at[pl.ds(row0, R)])
            else:
                for r in range(R):
                    for c in range(D_real // _L):
                        sl = pl.ds(c * _L, _L)
                        acc = gbuf[r * K, sl]
                        for kk in range(1, K):
                            acc = jnp.maximum(acc, gbuf[r * K + kk, sl])
                        outv[r, sl] = acc
                pltpu.sync_copy(outv, out_h.at[pl.ds(row0, R)])
            return carry

        lax.fori_loop(0, chunks, chunk_body, 0)

    return k(table, flat)[:Nout]


def kernel(feats, points0, neighbors0, points1, neighbors1, points2,
           neighbors2, points3, neighbors3, subsampling0, subsampling1,
           subsampling2, ups1, ups2, W_e11, W1a, W1b, Wsc1, W2a, W2b, W3a,
           W3b, W4a, W4b, W5a, W5b, W6a, W6b, Wsc6, W7a, W7b, Wsc7, Wd3, Wd2,
           Wq1, Wk1, Wv1, lin1W, lin1b, Wq2, Wk2, Wv2, lin2W, lin2b, Wq3,
           Wk3, Wv3, lin3W, lin3b):
    M64p = _gn_mat(64, 128)
    M128 = _gn_mat(128)
    M256 = _gn_mat(256)
    M512 = _gn_mat(512)
    M1024 = _gn_mat(1024)

    h0 = _tc_stage(feats, W=_cpad(W_e11))
    m0 = _sc_gather_max(h0, neighbors0, R=16, D_real=64)
    f0 = _tc_stage(m0, M=M64p, act=True)
    h1 = _tc_stage(f0, W=_rpad(_cpad(W1a)), M=M64p, act=True)
    m1 = _sc_gather_max(h1, neighbors0, R=16, D_real=64)
    f1 = _tc_stage(m1, W=_rpad(W1b), skip=f0, skipW=_rpad(Wsc1),
                   M=M128, act=True)
    h2 = _tc_stage(f1, W=_cpad(W2a), M=M64p, act=True)
    m2 = _sc_gather_max(h2, subsampling0, R=16, D_real=64)
    s2 = _sc_gather_max(f1, subsampling0, R=8)
    f2 = _tc_stage(m2, W=_rpad(W2b), skip=s2, M=M128, act=True)
    h3 = _tc_stage(f2, W=_cpad(W3a), M=M64p, act=True)
    m3 = _sc_gather_max(h3, neighbors1, R=16, D_real=64)
    f2 = _tc_stage(m3, W=_rpad(W3b), skip=f2, M=M128, act=True)
    h4 = _tc_stage(f2, W=_cpad(W4a), M=M64p, act=True)
    m4 = _sc_gather_max(h4, subsampling1, R=4, D_real=64)
    s4 = _sc_gather_max(f2, subsampling1, R=4)
    f3 = _tc_stage(m4, W=_rpad(W4b), skip=s4, M=M128, act=True)
    h5 = _tc_stage(f3, W=_cpad(W5a), M=M64p, act=True)
    m5 = _sc_gather_max(h5, neighbors2, R=4, D_real=64)
    f3 = _tc_stage(m5, W=_rpad(W5b), skip=f3, M=M128, act=True)
    h6 = _tc_stage(f3, W=W6a, M=M128, act=True)
    m6 = _sc_gather_max(h6, subsampling2, R=1)
    s6 = _sc_gather_max(f3, subsampling2, R=1)
    f4 = _tc_stage(m6, W=W6b, skip=s6, skipW=Wsc6, M=M256, act=True)
    h7 = _tc_stage(f4, W=W7a, M=M512, act=True)
    m7 = _sc_gather_max(h7, neighbors3, R=1)
    f4 = _tc_stage(m7, W=W7b, skip=f4, skipW=Wsc7, M=M1024, act=True)
    g8 = _sc_gather_max(f4, ups2[:, None], R=16)
    l3 = _tc_stage(g8, W=Wd3[:1024], skip=f3, skipW=Wd3[1024:],
                   M=M128, gn_post=True, act=True)
    g9 = _sc_gather_max(l3, ups1[:, None], R=16)
    l2 = _tc_stage(g9, W=Wd2[:128], skip=f2, skipW=Wd2[128:])
    return (l2, l3, f4)
